# Initial kernel scaffold; baseline (speedup 1.0000x reference)
import jax
import jax.numpy as jnp
from jax.experimental import pallas as pl


def kernel(x, table, proj_w):
    raise NotImplementedError("write your pallas kernel here")


# speedup vs baseline: 23.5760x; 23.5760x over previous
import functools

import jax
import jax.numpy as jnp
from jax import lax
from jax.experimental import pallas as pl
from jax.experimental.pallas import tpu as pltpu
from jax.experimental.pallas import tpu_sc as plsc

_B, _T, _C, _D = 64, 100, 256, 32
_N = _B * _T
_NC, _NS = 2, 16
_NW = _NC * _NS
_PPW = _N // _NW
_CB = 8


def _tc_body(table_ref, pw_ref, out_ref, g_ref):
    c0 = pl.program_id(0)

    @pl.when(c0 == 0)
    def _():
        t = table_ref[...]
        g_ref[...] = t * 0.5 * (1.0 + lax.erf(t * (2.0 ** -0.5)))

    g = g_ref[...]
    for i in range(_CB):
        w = pw_ref[i * _D:(i + 1) * _D, :]
        out_ref[:, i * _D:(i + 1) * _D] = jnp.dot(
            g, w, preferred_element_type=jnp.float32)


def _precompute_p(table, proj_w):
    grid = (_C // _CB,)
    return pl.pallas_call(
        _tc_body,
        grid=grid,
        in_specs=[
            pl.BlockSpec((_C, _D), lambda c: (0, 0)),
            pl.BlockSpec((_CB * _D, _D), lambda c: (c, 0)),
        ],
        out_specs=pl.BlockSpec((_C, _CB * _D), lambda c: (0, c)),
        out_shape=jax.ShapeDtypeStruct((_C, _C * _D), jnp.float32),
        scratch_shapes=[pltpu.VMEM((_C, _D), jnp.float32)],
    )(table, proj_w)


def _sc_body(p_hbm, ids_hbm, out_hbm, idx_v, buf_v, acc_v, sem0, sem1):
    wid = lax.axis_index("s") * _NC + lax.axis_index("c")
    base = wid * _PPW
    pltpu.sync_copy(ids_hbm.at[pl.ds(base * 2, _PPW * 2)], idx_v)

    def pos_body(p, _):
        cp0 = pltpu.async_copy(
            p_hbm.at[idx_v.at[2 * p]], buf_v.at[pl.ds(0, 128)], sem0)
        cp1 = pltpu.async_copy(
            p_hbm.at[idx_v.at[2 * p + 1]], buf_v.at[pl.ds(128, 128)], sem1)
        cp0.wait()
        cp1.wait()

        def red(i, accs):
            a0, a1 = accs
            r = i * 8
            for k in range(8):
                a0 = a0 + buf_v[r + k, pl.ds(0, 16)]
                a1 = a1 + buf_v[r + k, pl.ds(16, 16)]
            return (a0, a1)

        z = jnp.zeros((16,), jnp.float32)
        a0, a1 = lax.fori_loop(0, _C // 8, red, (z, z))
        acc_v[p, pl.ds(0, 16)] = a0
        acc_v[p, pl.ds(16, 16)] = a1
        return 0

    lax.fori_loop(0, _PPW, pos_body, 0)
    pltpu.sync_copy(acc_v, out_hbm.at[pl.ds(base, _PPW)])


@functools.partial(
    pl.kernel,
    mesh=plsc.VectorSubcoreMesh(core_axis_name="c", subcore_axis_name="s"),
    compiler_params=pltpu.CompilerParams(use_tc_tiling_on_sc=False),
    out_type=jax.ShapeDtypeStruct((_N, _D), jnp.float32),
    scratch_types=[
        pltpu.VMEM((2 * _PPW, 128), jnp.int32),
        pltpu.VMEM((_C, _D), jnp.float32),
        pltpu.VMEM((_PPW, _D), jnp.float32),
        pltpu.SemaphoreType.DMA,
        pltpu.SemaphoreType.DMA,
    ],
)
def _sc_gather_sum(p_hbm, ids_hbm, out_hbm, idx_v, buf_v, acc_v, sem0, sem1):
    _sc_body(p_hbm, ids_hbm, out_hbm, idx_v, buf_v, acc_v, sem0, sem1)


def kernel(x, table, proj_w):
    p2 = _precompute_p(table, proj_w)
    pflat = p2.reshape(_C * _C, _D)
    ids = (x.reshape(_N, _C) * _C
           + jnp.arange(_C, dtype=jnp.int32)[None, :]).reshape(2 * _N, 128)
    out = _sc_gather_sum(pflat, ids)
    return out.reshape(_B, _T, _D)


# speedup vs baseline: 34.2148x; 1.4513x over previous
import functools

import jax
import jax.numpy as jnp
from jax import lax
from jax.experimental import pallas as pl
from jax.experimental.pallas import tpu as pltpu
from jax.experimental.pallas import tpu_sc as plsc

_B, _T, _C, _D = 64, 100, 256, 32
_N = _B * _T
_NC, _NS = 2, 16
_NW = _NC * _NS
_PPW = _N // _NW
_CB = 8


def _tc_body(table_ref, pw_ref, out_ref, g_ref):
    c0 = pl.program_id(0)

    @pl.when(c0 == 0)
    def _():
        t = table_ref[...]
        g_ref[...] = t * 0.5 * (1.0 + lax.erf(t * (2.0 ** -0.5)))

    g = g_ref[...]
    for i in range(_CB):
        w = pw_ref[i * _D:(i + 1) * _D, :]
        out_ref[:, i * _D:(i + 1) * _D] = jnp.dot(
            g, w, preferred_element_type=jnp.float32)


def _precompute_p(table, proj_w):
    grid = (_C // _CB,)
    return pl.pallas_call(
        _tc_body,
        grid=grid,
        in_specs=[
            pl.BlockSpec((_C, _D), lambda c: (0, 0)),
            pl.BlockSpec((_CB * _D, _D), lambda c: (c, 0)),
        ],
        out_specs=pl.BlockSpec((_C, _CB * _D), lambda c: (0, c)),
        out_shape=jax.ShapeDtypeStruct((_C, _C * _D), jnp.float32),
        scratch_shapes=[pltpu.VMEM((_C, _D), jnp.float32)],
    )(table, proj_w)


def _issue(p_hbm, idx_v, p, buf, sem):
    pltpu.async_copy(p_hbm.at[idx_v.at[2 * p]], buf.at[pl.ds(0, 128)], sem)
    pltpu.async_copy(
        p_hbm.at[idx_v.at[2 * p + 1]], buf.at[pl.ds(128, 128)], sem)


def _drain(p_hbm, buf, sem):
    pltpu.make_async_copy(
        p_hbm.at[pl.ds(0, 128)], buf.at[pl.ds(0, 128)], sem).wait()
    pltpu.make_async_copy(
        p_hbm.at[pl.ds(0, 128)], buf.at[pl.ds(128, 128)], sem).wait()


def _reduce_into(buf, acc_v, p):
    def red(i, accs):
        a0, a1 = accs
        r = i * 8
        for k in range(8):
            a0 = a0 + buf[r + k, pl.ds(0, 16)]
            a1 = a1 + buf[r + k, pl.ds(16, 16)]
        return (a0, a1)

    z = jnp.zeros((16,), jnp.float32)
    a0, a1 = lax.fori_loop(0, _C // 8, red, (z, z))
    acc_v[p, pl.ds(0, 16)] = a0
    acc_v[p, pl.ds(16, 16)] = a1


def _sc_body(p_hbm, ids_hbm, out_hbm, idx_v, buf0, buf1, acc_v, sem0, sem1):
    wid = lax.axis_index("s") * _NC + lax.axis_index("c")
    base = wid * _PPW
    pltpu.sync_copy(ids_hbm.at[pl.ds(base * 2, _PPW * 2)], idx_v)

    _issue(p_hbm, idx_v, 0, buf0, sem0)

    def pos_body(i, _):
        p0 = 2 * i
        _issue(p_hbm, idx_v, p0 + 1, buf1, sem1)
        _drain(p_hbm, buf0, sem0)
        _reduce_into(buf0, acc_v, p0)

        @pl.when(i < _PPW // 2 - 1)
        def _():
            _issue(p_hbm, idx_v, p0 + 2, buf0, sem0)

        _drain(p_hbm, buf1, sem1)
        _reduce_into(buf1, acc_v, p0 + 1)
        return 0

    lax.fori_loop(0, _PPW // 2, pos_body, 0)
    pltpu.sync_copy(acc_v, out_hbm.at[pl.ds(base, _PPW)])


@functools.partial(
    pl.kernel,
    mesh=plsc.VectorSubcoreMesh(core_axis_name="c", subcore_axis_name="s"),
    compiler_params=pltpu.CompilerParams(use_tc_tiling_on_sc=False),
    out_type=jax.ShapeDtypeStruct((_N, _D), jnp.float32),
    scratch_types=[
        pltpu.VMEM((2 * _PPW, 128), jnp.int32),
        pltpu.VMEM((_C, _D), jnp.float32),
        pltpu.VMEM((_C, _D), jnp.float32),
        pltpu.VMEM((_PPW, _D), jnp.float32),
        pltpu.SemaphoreType.DMA,
        pltpu.SemaphoreType.DMA,
    ],
)
def _sc_gather_sum(p_hbm, ids_hbm, out_hbm, idx_v, buf0, buf1, acc_v,
                   sem0, sem1):
    _sc_body(p_hbm, ids_hbm, out_hbm, idx_v, buf0, buf1, acc_v, sem0, sem1)


def kernel(x, table, proj_w):
    p2 = _precompute_p(table, proj_w)
    pflat = p2.reshape(_C * _C, _D)
    ids = (x.reshape(_N, _C) * _C
           + jnp.arange(_C, dtype=jnp.int32)[None, :]).reshape(2 * _N, 128)
    out = _sc_gather_sum(pflat, ids)
    return out.reshape(_B, _T, _D)


# speedup vs baseline: 34.6074x; 1.0115x over previous
import functools

import jax
import jax.numpy as jnp
from jax import lax
from jax.experimental import pallas as pl
from jax.experimental.pallas import tpu as pltpu
from jax.experimental.pallas import tpu_sc as plsc

_B, _T, _C, _D = 64, 100, 256, 32
_N = _B * _T
_NC, _NS = 2, 16
_NW = _NC * _NS
_PPW = _N // _NW
_CB = 8


def _tc_body(table_ref, a_ref, out_ref):
    t = table_ref[...]
    g = t * 0.5 * (1.0 + lax.erf(t * (2.0 ** -0.5)))
    out_ref[...] = jnp.dot(
        g, a_ref[...], preferred_element_type=jnp.float32
    ).astype(jnp.bfloat16)


def _precompute_p(table, a_mat):
    return pl.pallas_call(
        _tc_body,
        out_shape=jax.ShapeDtypeStruct((_C, _C * _D), jnp.bfloat16),
    )(table, a_mat)


def _issue(p_hbm, idx_v, p, buf, sem):
    pltpu.async_copy(p_hbm.at[idx_v.at[2 * p]], buf.at[pl.ds(0, 128)], sem)
    pltpu.async_copy(
        p_hbm.at[idx_v.at[2 * p + 1]], buf.at[pl.ds(128, 128)], sem)


def _drain(p_hbm, buf, sem):
    pltpu.make_async_copy(
        p_hbm.at[pl.ds(0, 128)], buf.at[pl.ds(0, 128)], sem).wait()
    pltpu.make_async_copy(
        p_hbm.at[pl.ds(0, 128)], buf.at[pl.ds(128, 128)], sem).wait()


def _reduce_into(buf, acc_v, p):
    def red(i, accs):
        a0, a1 = accs
        r = i * 8
        for k in range(8):
            lo, hi = plsc.unpack(
                buf[r + k, pl.ds(0, 32)],
                format=plsc.PackFormat.INTERLEAVED,
                preferred_element_type=jnp.float32)
            a0 = a0 + lo
            a1 = a1 + hi
        return (a0, a1)

    z = jnp.zeros((16,), jnp.float32)
    a0, a1 = lax.fori_loop(0, _C // 8, red, (z, z))
    acc_v[p, pl.ds(0, 16)] = a0
    acc_v[p, pl.ds(16, 16)] = a1


def _sc_body(p_hbm, ids_hbm, out_hbm, idx_v, buf0, buf1, acc_v, sem0, sem1):
    wid = lax.axis_index("s") * _NC + lax.axis_index("c")
    base = wid * _PPW
    pltpu.sync_copy(ids_hbm.at[pl.ds(base * 2, _PPW * 2)], idx_v)

    _issue(p_hbm, idx_v, 0, buf0, sem0)

    def pos_body(i, _):
        p0 = 2 * i
        _issue(p_hbm, idx_v, p0 + 1, buf1, sem1)
        _drain(p_hbm, buf0, sem0)
        _reduce_into(buf0, acc_v, p0)

        @pl.when(i < _PPW // 2 - 1)
        def _():
            _issue(p_hbm, idx_v, p0 + 2, buf0, sem0)

        _drain(p_hbm, buf1, sem1)
        _reduce_into(buf1, acc_v, p0 + 1)
        return 0

    lax.fori_loop(0, _PPW // 2, pos_body, 0)
    pltpu.sync_copy(acc_v, out_hbm.at[pl.ds(base, _PPW)])


@functools.partial(
    pl.kernel,
    mesh=plsc.VectorSubcoreMesh(core_axis_name="c", subcore_axis_name="s"),
    compiler_params=pltpu.CompilerParams(
        use_tc_tiling_on_sc=False, needs_layout_passes=False),
    out_type=jax.ShapeDtypeStruct((_N, _D), jnp.float32),
    scratch_types=[
        pltpu.VMEM((2 * _PPW, 128), jnp.int32),
        pltpu.VMEM((_C, _D), jnp.bfloat16),
        pltpu.VMEM((_C, _D), jnp.bfloat16),
        pltpu.VMEM((_PPW, _D), jnp.float32),
        pltpu.SemaphoreType.DMA,
        pltpu.SemaphoreType.DMA,
    ],
)
def _sc_gather_sum(p_hbm, ids_hbm, out_hbm, idx_v, buf0, buf1, acc_v,
                   sem0, sem1):
    _sc_body(p_hbm, ids_hbm, out_hbm, idx_v, buf0, buf1, acc_v, sem0, sem1)


_COL_PERM = tuple(
    int(v) for m in range(_D // 2) for v in (m, m + _D // 2))


def kernel(x, table, proj_w):
    a_mat = (proj_w.reshape(_C, _D, _D)
             .transpose(1, 0, 2)[:, :, _COL_PERM]
             .reshape(_D, _C * _D))
    p2 = _precompute_p(table, a_mat)
    pflat = p2.reshape(_C * _C, _D)
    ids = (x.reshape(_N, _C) * _C
           + jnp.arange(_C, dtype=jnp.int32)[None, :]).reshape(2 * _N, 128)
    out = _sc_gather_sum(pflat, ids)
    return out.reshape(_B, _T, _D)


# speedup vs baseline: 37.4605x; 1.0824x over previous
import functools

import jax
import jax.numpy as jnp
from jax import lax
from jax.experimental import pallas as pl
from jax.experimental.pallas import tpu as pltpu
from jax.experimental.pallas import tpu_sc as plsc

_B, _T, _C, _D = 64, 100, 256, 32
_N = _B * _T
_NC, _NS = 2, 16
_NW = _NC * _NS
_PPW = _N // _NW
_CB = 8
_NBUF = 4


def _tc_body(table_ref, pwp_ref, out_ref, g_ref):
    c0 = pl.program_id(0)

    @pl.when(c0 == 0)
    def _():
        t = table_ref[...]
        g_ref[...] = t * 0.5 * (1.0 + lax.erf(t * (2.0 ** -0.5)))

    g = g_ref[...]
    gwide = jnp.concatenate([g] * _CB, axis=1)
    pw = pwp_ref[...]
    pwwide = jnp.concatenate([pw] * _CB, axis=1)
    ri = lax.broadcasted_iota(jnp.int32, (_CB * _D, _CB * _D), 0)
    ci = lax.broadcasted_iota(jnp.int32, (_CB * _D, _CB * _D), 1)
    rhs = jnp.where((ri // _D) == (ci // _D), pwwide, 0.0)
    res = jnp.dot(gwide, rhs, preferred_element_type=jnp.float32)
    resb = res.astype(jnp.bfloat16)
    for i in range(_CB):
        out_ref[i * _C:(i + 1) * _C, :] = resb[:, i * _D:(i + 1) * _D]


def _precompute_p(table, pwp):
    return pl.pallas_call(
        _tc_body,
        grid=(_C // _CB,),
        in_specs=[
            pl.BlockSpec((_C, _D), lambda c: (0, 0)),
            pl.BlockSpec((_CB * _D, _D), lambda c: (c, 0)),
        ],
        out_specs=pl.BlockSpec((_CB * _C, _D), lambda c: (c, 0)),
        out_shape=jax.ShapeDtypeStruct((_C * _C, _D), jnp.bfloat16),
        scratch_shapes=[pltpu.VMEM((_C, _D), jnp.float32)],
    )(table, pwp)


def _issue(p_hbm, idx_v, p, buf, sem):
    pltpu.async_copy(p_hbm.at[idx_v.at[2 * p]], buf.at[pl.ds(0, 128)], sem)
    pltpu.async_copy(
        p_hbm.at[idx_v.at[2 * p + 1]], buf.at[pl.ds(128, 128)], sem)


def _drain(p_hbm, buf, sem):
    pltpu.make_async_copy(
        p_hbm.at[pl.ds(0, 128)], buf.at[pl.ds(0, 128)], sem).wait()
    pltpu.make_async_copy(
        p_hbm.at[pl.ds(0, 128)], buf.at[pl.ds(128, 128)], sem).wait()


def _reduce_into(buf, acc_v, p):
    def red(i, accs):
        a0, a1 = accs
        r = i * 8
        for k in range(8):
            lo, hi = plsc.unpack(
                buf[r + k, pl.ds(0, 32)],
                format=plsc.PackFormat.INTERLEAVED,
                preferred_element_type=jnp.float32)
            a0 = a0 + lo
            a1 = a1 + hi
        return (a0, a1)

    z = jnp.zeros((16,), jnp.float32)
    a0, a1 = lax.fori_loop(0, _C // 8, red, (z, z))
    acc_v[p, pl.ds(0, 16)] = a0
    acc_v[p, pl.ds(16, 16)] = a1


def _sc_body(p_hbm, ids_hbm, out_hbm, idx_v, bufs, acc_v, sems):
    wid = lax.axis_index("s") * _NC + lax.axis_index("c")
    base = wid * _PPW
    pltpu.sync_copy(ids_hbm.at[pl.ds(base * 2, _PPW * 2)], idx_v)

    for b in range(_NBUF - 1):
        _issue(p_hbm, idx_v, b, bufs[b], sems[b])

    def quad(i, _):
        p0 = i * _NBUF
        for b in range(_NBUF):
            p = p0 + b
            _drain(p_hbm, bufs[b], sems[b])
            _reduce_into(bufs[b], acc_v, p)

            @pl.when(p + _NBUF - 1 < _PPW)
            def _(p=p, b=b):
                _issue(p_hbm, idx_v, p + _NBUF - 1,
                       bufs[(b + _NBUF - 1) % _NBUF],
                       sems[(b + _NBUF - 1) % _NBUF])
        return 0

    lax.fori_loop(0, _PPW // _NBUF, quad, 0)
    pltpu.sync_copy(acc_v, out_hbm.at[pl.ds(base, _PPW)])


@functools.partial(
    pl.kernel,
    mesh=plsc.VectorSubcoreMesh(core_axis_name="c", subcore_axis_name="s"),
    compiler_params=pltpu.CompilerParams(
        use_tc_tiling_on_sc=False, needs_layout_passes=False),
    out_type=jax.ShapeDtypeStruct((_N, _D), jnp.float32),
    scratch_types=[
        pltpu.VMEM((2 * _PPW, 128), jnp.int32),
        pltpu.VMEM((_C, _D), jnp.bfloat16),
        pltpu.VMEM((_C, _D), jnp.bfloat16),
        pltpu.VMEM((_C, _D), jnp.bfloat16),
        pltpu.VMEM((_C, _D), jnp.bfloat16),
        pltpu.VMEM((_PPW, _D), jnp.float32),
        pltpu.SemaphoreType.DMA,
        pltpu.SemaphoreType.DMA,
        pltpu.SemaphoreType.DMA,
        pltpu.SemaphoreType.DMA,
    ],
)
def _sc_gather_sum(p_hbm, ids_hbm, out_hbm, idx_v, b0, b1, b2, b3, acc_v,
                   s0, s1, s2, s3):
    _sc_body(p_hbm, ids_hbm, out_hbm, idx_v,
             (b0, b1, b2, b3), acc_v, (s0, s1, s2, s3))


_COL_PERM = tuple(
    int(v) for m in range(_D // 2) for v in (m, m + _D // 2))


def kernel(x, table, proj_w):
    pwp = proj_w[:, _COL_PERM]
    pflat = _precompute_p(table, pwp)
    ids = (x.reshape(_N, _C)
           + (jnp.arange(_C, dtype=jnp.int32) * _C)[None, :]
           ).reshape(2 * _N, 128)
    out = _sc_gather_sum(pflat, ids)
    return out.reshape(_B, _T, _D)


# speedup vs baseline: 38.6882x; 1.0328x over previous
import functools

import jax
import jax.numpy as jnp
from jax import lax
from jax.experimental import pallas as pl
from jax.experimental.pallas import tpu as pltpu
from jax.experimental.pallas import tpu_sc as plsc

_B, _T, _C, _D = 64, 100, 256, 32
_N = _B * _T
_NC, _NS = 2, 16
_NW = _NC * _NS
_PPW = _N // _NW
_BPW = _PPW // _T
_CB = 8
_SUB = 4
_NBUF = 4


def _tc_body(table_ref, pwp_ref, out_ref, gw_ref, mask_ref):
    c0 = pl.program_id(0)
    k = _CB * _D

    @pl.when(c0 == 0)
    def _():
        t = table_ref[...]
        g = t * 0.5 * (1.0 + lax.erf(t * (2.0 ** -0.5)))
        gw_ref[...] = jnp.concatenate([g] * _CB, axis=1).astype(jnp.bfloat16)
        ri = lax.broadcasted_iota(jnp.int32, (k, k), 0)
        ci = lax.broadcasted_iota(jnp.int32, (k, k), 1)
        mask_ref[...] = jnp.where((ri // _D) == (ci // _D), 1.0, 0.0)

    gw = gw_ref[...]
    mask = mask_ref[...]
    for s in range(_SUB):
        pw = pwp_ref[s * k:(s + 1) * k, :]
        pwwide = jnp.concatenate([pw] * _CB, axis=1)
        rhs = (pwwide * mask).astype(jnp.bfloat16)
        res = jnp.dot(gw, rhs, preferred_element_type=jnp.float32)
        resb = res.astype(jnp.bfloat16)
        for i in range(_CB):
            out_ref[(s * _CB + i) * _C:(s * _CB + i + 1) * _C, :] = (
                resb[:, i * _D:(i + 1) * _D])


def _precompute_p(table, pwp):
    steps = _C // (_CB * _SUB)
    return pl.pallas_call(
        _tc_body,
        grid=(steps,),
        in_specs=[
            pl.BlockSpec((_C, _D), lambda c: (0, 0)),
            pl.BlockSpec((_SUB * _CB * _D, _D), lambda c: (c, 0)),
        ],
        out_specs=pl.BlockSpec((_SUB * _CB * _C, _D), lambda c: (c, 0)),
        out_shape=jax.ShapeDtypeStruct((_C * _C, _D), jnp.bfloat16),
        scratch_shapes=[
            pltpu.VMEM((_C, _CB * _D), jnp.bfloat16),
            pltpu.VMEM((_CB * _D, _CB * _D), jnp.float32),
        ],
    )(table, pwp)


def _issue(p_hbm, idx_v, p, buf, sem):
    b1 = p // _T
    t = p - b1 * _T
    pltpu.async_copy(
        p_hbm.at[idx_v.at[b1, t, pl.ds(0, 128)]], buf.at[pl.ds(0, 128)], sem)
    pltpu.async_copy(
        p_hbm.at[idx_v.at[b1, t, pl.ds(128, 128)]],
        buf.at[pl.ds(128, 128)], sem)


def _drain(p_hbm, buf, sem):
    pltpu.make_async_copy(
        p_hbm.at[pl.ds(0, 128)], buf.at[pl.ds(0, 128)], sem).wait()
    pltpu.make_async_copy(
        p_hbm.at[pl.ds(0, 128)], buf.at[pl.ds(128, 128)], sem).wait()


def _reduce_into(buf, acc_v, p):
    def red(i, accs):
        a0, a1 = accs
        r = i * 8
        for k in range(8):
            lo, hi = plsc.unpack(
                buf[r + k, pl.ds(0, 32)],
                format=plsc.PackFormat.INTERLEAVED,
                preferred_element_type=jnp.float32)
            a0 = a0 + lo
            a1 = a1 + hi
        return (a0, a1)

    z = jnp.zeros((16,), jnp.float32)
    a0, a1 = lax.fori_loop(0, _C // 8, red, (z, z))
    acc_v[p, pl.ds(0, 16)] = a0
    acc_v[p, pl.ds(16, 16)] = a1


def _sc_body(p_hbm, ids_hbm, out_hbm, idx_v, bufs, acc_v, sems):
    wid = lax.axis_index("s") * _NC + lax.axis_index("c")
    base = wid * _PPW
    pltpu.sync_copy(ids_hbm.at[pl.ds(wid * _BPW, _BPW)], idx_v)

    for b in range(_NBUF - 1):
        _issue(p_hbm, idx_v, b, bufs[b], sems[b])

    def ring(i, _):
        p0 = i * _NBUF
        for b in range(_NBUF):
            p = p0 + b
            _drain(p_hbm, bufs[b], sems[b])
            _reduce_into(bufs[b], acc_v, p)

            @pl.when(p + _NBUF - 1 < _PPW)
            def _(p=p, b=b):
                _issue(p_hbm, idx_v, p + _NBUF - 1,
                       bufs[(b + _NBUF - 1) % _NBUF],
                       sems[(b + _NBUF - 1) % _NBUF])
        return 0

    lax.fori_loop(0, _PPW // _NBUF, ring, 0)
    pltpu.sync_copy(acc_v, out_hbm.at[pl.ds(base, _PPW)])


@functools.partial(
    pl.kernel,
    mesh=plsc.VectorSubcoreMesh(core_axis_name="c", subcore_axis_name="s"),
    compiler_params=pltpu.CompilerParams(
        use_tc_tiling_on_sc=False, needs_layout_passes=False),
    out_type=jax.ShapeDtypeStruct((_N, _D), jnp.float32),
    scratch_types=[
        pltpu.VMEM((_BPW, _T, _C), jnp.int32),
        pltpu.VMEM((_C, _D), jnp.bfloat16),
        pltpu.VMEM((_C, _D), jnp.bfloat16),
        pltpu.VMEM((_C, _D), jnp.bfloat16),
        pltpu.VMEM((_C, _D), jnp.bfloat16),
        pltpu.VMEM((_PPW, _D), jnp.float32),
        pltpu.SemaphoreType.DMA,
        pltpu.SemaphoreType.DMA,
        pltpu.SemaphoreType.DMA,
        pltpu.SemaphoreType.DMA,
    ],
)
def _sc_gather_sum(p_hbm, ids_hbm, out_hbm, idx_v, b0, b1, b2, b3, acc_v,
                   s0, s1, s2, s3):
    _sc_body(p_hbm, ids_hbm, out_hbm, idx_v,
             (b0, b1, b2, b3), acc_v, (s0, s1, s2, s3))


_COL_PERM = tuple(
    int(v) for m in range(_D // 2) for v in (m, m + _D // 2))


def kernel(x, table, proj_w):
    pwp = proj_w[:, _COL_PERM]
    pflat = _precompute_p(table, pwp)
    ids = x + (jnp.arange(_C, dtype=jnp.int32) * _C)[None, None, :]
    out = _sc_gather_sum(pflat, ids)
    return out.reshape(_B, _T, _D)


# speedup vs baseline: 48.6513x; 1.2575x over previous
import functools

import jax
import jax.numpy as jnp
from jax import lax
from jax.experimental import pallas as pl
from jax.experimental.pallas import tpu as pltpu
from jax.experimental.pallas import tpu_sc as plsc

_B, _T, _C, _D = 64, 100, 256, 32
_N = _B * _T
_NC, _NS = 2, 16
_NW = _NC * _NS
_PPW = _N // _NW
_BPW = _PPW // _T
_CB = 8
_SUB = 4
_NBUF = 4


def _tc_body(table_ref, pw_ref, out_ref, gw_ref, mask_ref):
    c0 = pl.program_id(0)
    k = _CB * _D

    @pl.when(c0 == 0)
    def _():
        t = table_ref[...]
        g = t * 0.5 * (1.0 + lax.erf(t * (2.0 ** -0.5)))
        gw_ref[...] = jnp.concatenate([g] * _CB, axis=1).astype(jnp.bfloat16)
        ri = lax.broadcasted_iota(jnp.int32, (k, k), 0)
        ci = lax.broadcasted_iota(jnp.int32, (k, k), 1)
        mask_ref[...] = jnp.where((ri // _D) == (ci // _D), 1.0, 0.0)

    gw = gw_ref[...]
    mask = mask_ref[...]
    for s in range(_SUB):
        pw = pw_ref[s * k:(s + 1) * k, :]
        pwwide = jnp.concatenate([pw] * _CB, axis=1)
        rhs = (pwwide * mask).astype(jnp.bfloat16)
        res = jnp.dot(gw, rhs, preferred_element_type=jnp.float32)
        out_ref[s * 512:s * 512 + 256, :] = res[:, :128]
        out_ref[s * 512 + 256:s * 512 + 512, :] = res[:, 128:]


def _precompute_p(table, proj_w):
    steps = _C // (_CB * _SUB)
    return pl.pallas_call(
        _tc_body,
        grid=(steps,),
        in_specs=[
            pl.BlockSpec((_C, _D), lambda c: (0, 0)),
            pl.BlockSpec((_SUB * _CB * _D, _D), lambda c: (c, 0)),
        ],
        out_specs=pl.BlockSpec((_SUB * 512, 128), lambda c: (c, 0)),
        out_shape=jax.ShapeDtypeStruct((_C * _C // 4, 128), jnp.float32),
        scratch_shapes=[
            pltpu.VMEM((_C, _CB * _D), jnp.bfloat16),
            pltpu.VMEM((_CB * _D, _CB * _D), jnp.float32),
        ],
    )(table, proj_w)


def _issue(p_hbm, idx_v, p, buf, sem):
    b1 = p // _T
    t = p - b1 * _T
    pltpu.async_copy(
        p_hbm.at[idx_v.at[b1, t, pl.ds(0, 128)]], buf.at[pl.ds(0, 128)], sem)
    pltpu.async_copy(
        p_hbm.at[idx_v.at[b1, t, pl.ds(128, 128)]],
        buf.at[pl.ds(128, 128)], sem)


def _drain(p_hbm, buf, sem):
    pltpu.make_async_copy(
        p_hbm.at[pl.ds(0, 128)], buf.at[pl.ds(0, 128)], sem).wait()
    pltpu.make_async_copy(
        p_hbm.at[pl.ds(0, 128)], buf.at[pl.ds(128, 128)], sem).wait()


def _reduce_into(buf, acc_v, p):
    def red(i, accs):
        a0, a1 = accs
        r = i * 8
        for k in range(8):
            a0 = a0 + buf[r + k, pl.ds(0, 16)]
            a1 = a1 + buf[r + k, pl.ds(16, 16)]
        return (a0, a1)

    z = jnp.zeros((16,), jnp.float32)
    a0, a1 = lax.fori_loop(0, _C // 8, red, (z, z))
    acc_v[p, pl.ds(0, 16)] = a0
    acc_v[p, pl.ds(16, 16)] = a1


def _sc_body(p_hbm, ids_hbm, out_hbm, idx_v, bufs, acc_v, sems):
    wid = lax.axis_index("s") * _NC + lax.axis_index("c")
    base = wid * _PPW
    pltpu.sync_copy(ids_hbm.at[pl.ds(wid * _BPW, _BPW)], idx_v)

    for b in range(_NBUF - 1):
        _issue(p_hbm, idx_v, b, bufs[b], sems[b])

    def ring(i, _):
        p0 = i * _NBUF
        for b in range(_NBUF):
            p = p0 + b
            _drain(p_hbm, bufs[b], sems[b])
            _reduce_into(bufs[b], acc_v, p)

            @pl.when(p + _NBUF - 1 < _PPW)
            def _(p=p, b=b):
                _issue(p_hbm, idx_v, p + _NBUF - 1,
                       bufs[(b + _NBUF - 1) % _NBUF],
                       sems[(b + _NBUF - 1) % _NBUF])
        return 0

    lax.fori_loop(0, _PPW // _NBUF, ring, 0)
    pltpu.sync_copy(acc_v, out_hbm.at[pl.ds(base, _PPW)])


@functools.partial(
    pl.kernel,
    mesh=plsc.VectorSubcoreMesh(core_axis_name="c", subcore_axis_name="s"),
    compiler_params=pltpu.CompilerParams(
        use_tc_tiling_on_sc=False, needs_layout_passes=False),
    out_type=jax.ShapeDtypeStruct((_N, _D), jnp.float32),
    scratch_types=[
        pltpu.VMEM((_BPW, _T, _C), jnp.int32),
        pltpu.VMEM((_C, _D), jnp.float32),
        pltpu.VMEM((_C, _D), jnp.float32),
        pltpu.VMEM((_C, _D), jnp.float32),
        pltpu.VMEM((_C, _D), jnp.float32),
        pltpu.VMEM((_PPW, _D), jnp.float32),
        pltpu.SemaphoreType.DMA,
        pltpu.SemaphoreType.DMA,
        pltpu.SemaphoreType.DMA,
        pltpu.SemaphoreType.DMA,
    ],
)
def _sc_gather_sum(p_hbm, ids_hbm, out_hbm, idx_v, b0, b1, b2, b3, acc_v,
                   s0, s1, s2, s3):
    _sc_body(p_hbm, ids_hbm, out_hbm, idx_v,
             (b0, b1, b2, b3), acc_v, (s0, s1, s2, s3))


def _offs():
    c = jnp.arange(_C, dtype=jnp.int32)
    return (c // 8) * 2048 + ((c % 8) // 4) * 1024 + (c % 4)


def kernel(x, table, proj_w):
    ptc = _precompute_p(table, proj_w)
    pflat = ptc.reshape(_C * _C, _D)
    ids = x * 4 + _offs()[None, None, :]
    out = _sc_gather_sum(pflat, ids)
    return out.reshape(_B, _T, _D)


# speedup vs baseline: 51.7138x; 1.0629x over previous
import functools

import jax
import jax.numpy as jnp
from jax import lax
from jax.experimental import pallas as pl
from jax.experimental.pallas import tpu as pltpu
from jax.experimental.pallas import tpu_sc as plsc

_B, _T, _C, _D = 64, 100, 256, 32
_N = _B * _T
_NC, _NS = 2, 16
_NW = _NC * _NS
_PPW = _N // _NW
_BPW = _PPW // _T
_CB = 8
_SUB = 4
_NBUF = 8


def _tc_body(table_ref, pw_ref, out_ref, gw_ref, mask_ref):
    c0 = pl.program_id(0)
    k = _CB * _D

    @pl.when(c0 == 0)
    def _():
        t = table_ref[...]
        g = t * 0.5 * (1.0 + lax.erf(t * (2.0 ** -0.5)))
        gw_ref[...] = jnp.concatenate([g] * _CB, axis=1).astype(jnp.bfloat16)
        ri = lax.broadcasted_iota(jnp.int32, (k, k), 0)
        ci = lax.broadcasted_iota(jnp.int32, (k, k), 1)
        mask_ref[...] = jnp.where((ri // _D) == (ci // _D), 1.0, 0.0)

    gw = gw_ref[...]
    mask = mask_ref[...]
    for s in range(_SUB):
        pw = pw_ref[s * k:(s + 1) * k, :]
        pwwide = jnp.concatenate([pw] * _CB, axis=1)
        rhs = (pwwide * mask).astype(jnp.bfloat16)
        res = jnp.dot(gw, rhs, preferred_element_type=jnp.float32)
        out_ref[s * 512:s * 512 + 256, :] = res[:, :128]
        out_ref[s * 512 + 256:s * 512 + 512, :] = res[:, 128:]


def _precompute_p(table, proj_w):
    steps = _C // (_CB * _SUB)
    return pl.pallas_call(
        _tc_body,
        grid=(steps,),
        in_specs=[
            pl.BlockSpec((_C, _D), lambda c: (0, 0)),
            pl.BlockSpec((_SUB * _CB * _D, _D), lambda c: (c, 0)),
        ],
        out_specs=pl.BlockSpec((_SUB * 512, 128), lambda c: (c, 0)),
        out_shape=jax.ShapeDtypeStruct((_C * _C // 4, 128), jnp.float32),
        scratch_shapes=[
            pltpu.VMEM((_C, _CB * _D), jnp.bfloat16),
            pltpu.VMEM((_CB * _D, _CB * _D), jnp.float32),
        ],
    )(table, proj_w)


def _issue(p_hbm, idx_v, p, buf, sem):
    pltpu.async_copy(p_hbm.at[idx_v.at[2 * p]], buf.at[pl.ds(0, 128)], sem)
    pltpu.async_copy(
        p_hbm.at[idx_v.at[2 * p + 1]], buf.at[pl.ds(128, 128)], sem)


def _drain(p_hbm, buf, sem):
    pltpu.make_async_copy(
        p_hbm.at[pl.ds(0, 128)], buf.at[pl.ds(0, 128)], sem).wait()
    pltpu.make_async_copy(
        p_hbm.at[pl.ds(0, 128)], buf.at[pl.ds(128, 128)], sem).wait()


def _reduce_into(buf, acc_v, p):
    def red(i, accs):
        a0, a1 = accs
        r = i * 8
        for k in range(8):
            a0 = a0 + buf[r + k, pl.ds(0, 16)]
            a1 = a1 + buf[r + k, pl.ds(16, 16)]
        return (a0, a1)

    z = jnp.zeros((16,), jnp.float32)
    a0, a1 = lax.fori_loop(0, _C // 8, red, (z, z))
    acc_v[p, pl.ds(0, 16)] = a0
    acc_v[p, pl.ds(16, 16)] = a1


def _sc_body(p_hbm, ids_hbm, out_hbm, idx_v, bufs, acc_v, sems):
    wid = lax.axis_index("s") * _NC + lax.axis_index("c")
    base = wid * _PPW
    pltpu.sync_copy(ids_hbm.at[pl.ds(base * 2, _PPW * 2)], idx_v)

    for b in range(_NBUF - 1):
        _issue(p_hbm, idx_v, b, bufs[b], sems[b])

    def ring(i, _):
        p0 = i * _NBUF
        for b in range(_NBUF):
            p = p0 + b
            _drain(p_hbm, bufs[b], sems[b])
            _reduce_into(bufs[b], acc_v, p)

            @pl.when(p + _NBUF - 1 < _PPW)
            def _(p=p, b=b):
                _issue(p_hbm, idx_v, p + _NBUF - 1,
                       bufs[(b + _NBUF - 1) % _NBUF],
                       sems[(b + _NBUF - 1) % _NBUF])
        return 0

    lax.fori_loop(0, _PPW // _NBUF, ring, 0)
    pltpu.sync_copy(acc_v, out_hbm.at[pl.ds(base, _PPW)])


@functools.partial(
    pl.kernel,
    mesh=plsc.VectorSubcoreMesh(core_axis_name="c", subcore_axis_name="s"),
    compiler_params=pltpu.CompilerParams(
        use_tc_tiling_on_sc=False, needs_layout_passes=False),
    out_type=jax.ShapeDtypeStruct((_N, _D), jnp.float32),
    scratch_types=(
        [pltpu.VMEM((2 * _PPW, 128), jnp.int32)]
        + [pltpu.VMEM((_C, _D), jnp.float32) for _ in range(_NBUF)]
        + [pltpu.VMEM((_PPW, _D), jnp.float32)]
        + [pltpu.SemaphoreType.DMA for _ in range(_NBUF)]
    ),
)
def _sc_gather_sum(p_hbm, ids_hbm, out_hbm, idx_v, *rest):
    bufs = rest[:_NBUF]
    acc_v = rest[_NBUF]
    sems = rest[_NBUF + 1:]
    _sc_body(p_hbm, ids_hbm, out_hbm, idx_v, bufs, acc_v, sems)


def _offs():
    c = jnp.arange(_C, dtype=jnp.int32)
    return (c // 8) * 2048 + ((c % 8) // 4) * 1024 + (c % 4)


def kernel(x, table, proj_w):
    ptc = _precompute_p(table, proj_w)
    pflat = ptc.reshape(_C * _C, _D)
    ids = (x.reshape(_N, _C) * 4 + _offs()[None, :]).reshape(2 * _N, 128)
    out = _sc_gather_sum(pflat, ids)
    return out.reshape(_B, _T, _D)


# speedup vs baseline: 56.5186x; 1.0929x over previous
import functools

import jax
import jax.numpy as jnp
from jax import lax
from jax.experimental import pallas as pl
from jax.experimental.pallas import tpu as pltpu
from jax.experimental.pallas import tpu_sc as plsc

_B, _T, _C, _D = 64, 100, 256, 32
_N = _B * _T
_NC, _NS = 2, 16
_NW = _NC * _NS
_PPW = _N // _NW
_BPW = _PPW // _T
_CB = 8
_SUB = 4
_NBUF = 8


def _tc_body(table_ref, pw_ref, out_ref, gw_ref, mask_ref):
    c0 = pl.program_id(0)
    k = _CB * _D

    @pl.when(c0 == 0)
    def _():
        t = table_ref[...]
        g = t * 0.5 * (1.0 + lax.erf(t * (2.0 ** -0.5)))
        gw_ref[...] = jnp.concatenate([g] * _CB, axis=1).astype(jnp.bfloat16)
        ri = lax.broadcasted_iota(jnp.int32, (k, k), 0)
        ci = lax.broadcasted_iota(jnp.int32, (k, k), 1)
        mask_ref[...] = jnp.where((ri // _D) == (ci // _D), 1.0, 0.0)

    gw = gw_ref[...]
    mask = mask_ref[...]
    for s in range(_SUB):
        pw = pw_ref[s * k:(s + 1) * k, :]
        pwwide = jnp.concatenate([pw] * _CB, axis=1)
        rhs = (pwwide * mask).astype(jnp.bfloat16)
        res = jnp.dot(gw, rhs, preferred_element_type=jnp.float32)
        out_ref[s * 512:s * 512 + 256, :] = res[:, :128]
        out_ref[s * 512 + 256:s * 512 + 512, :] = res[:, 128:]


def _precompute_p(table, proj_w):
    steps = _C // (_CB * _SUB)
    return pl.pallas_call(
        _tc_body,
        grid=(steps,),
        in_specs=[
            pl.BlockSpec((_C, _D), lambda c: (0, 0)),
            pl.BlockSpec((_SUB * _CB * _D, _D), lambda c: (c, 0)),
        ],
        out_specs=pl.BlockSpec((_SUB * 512, 128), lambda c: (c, 0)),
        out_shape=jax.ShapeDtypeStruct((_C * _C // 4, 128), jnp.float32),
        scratch_shapes=[
            pltpu.VMEM((_C, _CB * _D), jnp.bfloat16),
            pltpu.VMEM((_CB * _D, _CB * _D), jnp.float32),
        ],
    )(table, proj_w)


def _issue(p_hbm, idx_v, p, buf, sem):
    pltpu.async_copy(
        p_hbm.at[idx_v.at[p, pl.ds(0, 128)]], buf.at[pl.ds(0, 128)], sem)
    pltpu.async_copy(
        p_hbm.at[idx_v.at[p, pl.ds(128, 128)]], buf.at[pl.ds(128, 128)], sem)


def _drain(p_hbm, buf, sem):
    pltpu.make_async_copy(
        p_hbm.at[pl.ds(0, 128)], buf.at[pl.ds(0, 128)], sem).wait()
    pltpu.make_async_copy(
        p_hbm.at[pl.ds(0, 128)], buf.at[pl.ds(128, 128)], sem).wait()


def _reduce_into(buf, acc_v, p):
    def red(i, accs):
        a0, a1 = accs
        r = i * 8
        for k in range(8):
            a0 = a0 + buf[r + k, pl.ds(0, 16)]
            a1 = a1 + buf[r + k, pl.ds(16, 16)]
        return (a0, a1)

    z = jnp.zeros((16,), jnp.float32)
    a0, a1 = lax.fori_loop(0, _C // 8, red, (z, z))
    b1 = p // _T
    t = p - b1 * _T
    acc_v[b1, t, pl.ds(0, 16)] = a0
    acc_v[b1, t, pl.ds(16, 16)] = a1


def _sc_body(p_hbm, x_hbm, out_hbm, idx_v, bufs, acc_v, sems):
    wid = lax.axis_index("s") * _NC + lax.axis_index("c")
    for j2 in range(_BPW):
        pltpu.sync_copy(
            x_hbm.at[wid * _BPW + j2], idx_v.at[pl.ds(j2 * _T, _T)])

    lane = lax.iota(jnp.int32, 16)
    patt = ((lane // 8) * 2048 + ((lane % 8) // 4) * 1024 + (lane % 4))

    def xform(r, _):
        for j in range(_C // 16):
            v = idx_v[r, pl.ds(16 * j, 16)]
            idx_v[r, pl.ds(16 * j, 16)] = v * 4 + (patt + 4096 * j)
        return 0

    lax.fori_loop(0, _PPW, xform, 0)

    for b in range(_NBUF - 1):
        _issue(p_hbm, idx_v, b, bufs[b], sems[b])

    def ring(i, _):
        p0 = i * _NBUF
        for b in range(_NBUF):
            p = p0 + b
            _drain(p_hbm, bufs[b], sems[b])
            _reduce_into(bufs[b], acc_v, p)

            @pl.when(p + _NBUF - 1 < _PPW)
            def _(p=p, b=b):
                _issue(p_hbm, idx_v, p + _NBUF - 1,
                       bufs[(b + _NBUF - 1) % _NBUF],
                       sems[(b + _NBUF - 1) % _NBUF])
        return 0

    lax.fori_loop(0, _PPW // _NBUF, ring, 0)
    pltpu.sync_copy(acc_v, out_hbm.at[pl.ds(wid * _BPW, _BPW)])


@functools.partial(
    pl.kernel,
    mesh=plsc.VectorSubcoreMesh(core_axis_name="c", subcore_axis_name="s"),
    compiler_params=pltpu.CompilerParams(
        use_tc_tiling_on_sc=False, needs_layout_passes=False),
    out_type=jax.ShapeDtypeStruct((_B, _T, _D), jnp.float32),
    scratch_types=(
        [pltpu.VMEM((_PPW, _C), jnp.int32)]
        + [pltpu.VMEM((_C, _D), jnp.float32) for _ in range(_NBUF)]
        + [pltpu.VMEM((_BPW, _T, _D), jnp.float32)]
        + [pltpu.SemaphoreType.DMA for _ in range(_NBUF)]
    ),
)
def _sc_gather_sum(p_hbm, ids_hbm, out_hbm, idx_v, *rest):
    bufs = rest[:_NBUF]
    acc_v = rest[_NBUF]
    sems = rest[_NBUF + 1:]
    _sc_body(p_hbm, ids_hbm, out_hbm, idx_v, bufs, acc_v, sems)


def kernel(x, table, proj_w):
    ptc = _precompute_p(table, proj_w)
    pflat = ptc.reshape(_C * _C, _D)
    return _sc_gather_sum(pflat, x)


# speedup vs baseline: 58.4251x; 1.0337x over previous
import functools

import jax
import jax.numpy as jnp
from jax import lax
from jax.experimental import pallas as pl
from jax.experimental.pallas import tpu as pltpu
from jax.experimental.pallas import tpu_sc as plsc

_B, _T, _C, _D = 64, 100, 256, 32
_N = _B * _T
_NC, _NS = 2, 16
_NW = _NC * _NS
_PPW = _N // _NW
_BPW = _PPW // _T
_CB = 8
_SUB = 4
_NBUF = 8


def _tc_body(table_ref, pw_ref, out_ref, gw_ref, mask_ref):
    c0 = pl.program_id(0)
    k = _CB * _D

    @pl.when(c0 == 0)
    def _():
        t = table_ref[...]
        g = t * 0.5 * (1.0 + lax.erf(t * (2.0 ** -0.5)))
        gw_ref[...] = jnp.concatenate([g] * _CB, axis=1).astype(jnp.bfloat16)
        ri = lax.broadcasted_iota(jnp.int32, (k, k), 0)
        ci = lax.broadcasted_iota(jnp.int32, (k, k), 1)
        mask_ref[...] = jnp.where((ri // _D) == (ci // _D), 1.0, 0.0)

    gw = gw_ref[...]
    mask = mask_ref[...]
    for s in range(_SUB):
        pw = pw_ref[s * k:(s + 1) * k, :]
        pwwide = jnp.concatenate([pw] * _CB, axis=1)
        rhs = (pwwide * mask).astype(jnp.bfloat16)
        res = jnp.dot(gw, rhs, preferred_element_type=jnp.float32)
        ri = lax.bitcast_convert_type(res, jnp.int32)
        rnd = lax.shift_right_logical(
            ri + 0x7FFF + (lax.shift_right_logical(ri, 16) & 1), 16)
        parts = []
        for c8 in range(_CB):
            a = rnd[:, c8 * _D:c8 * _D + 16]
            b = rnd[:, c8 * _D + 16:c8 * _D + 32]
            parts.append(a | (b << 16))
        words = jnp.concatenate(parts, axis=1)
        out_ref[s * 256:(s + 1) * 256, :] = lax.bitcast_convert_type(
            words, jnp.float32)


def _precompute_p(table, proj_w):
    steps = _C // (_CB * _SUB)
    return pl.pallas_call(
        _tc_body,
        grid=(steps,),
        in_specs=[
            pl.BlockSpec((_C, _D), lambda c: (0, 0)),
            pl.BlockSpec((_SUB * _CB * _D, _D), lambda c: (c, 0)),
        ],
        out_specs=pl.BlockSpec((_SUB * 256, 128), lambda c: (c, 0)),
        out_shape=jax.ShapeDtypeStruct((_C * _C // 8, 128), jnp.float32),
        scratch_shapes=[
            pltpu.VMEM((_C, _CB * _D), jnp.bfloat16),
            pltpu.VMEM((_CB * _D, _CB * _D), jnp.float32),
        ],
    )(table, proj_w)


def _issue(p_hbm, idx_v, p, buf, sem):
    pltpu.async_copy(
        p_hbm.at[idx_v.at[p, pl.ds(0, 128)]], buf.at[pl.ds(0, 128)], sem)
    pltpu.async_copy(
        p_hbm.at[idx_v.at[p, pl.ds(128, 128)]], buf.at[pl.ds(128, 128)], sem)


def _drain(p_hbm, buf, sem):
    pltpu.make_async_copy(
        p_hbm.at[pl.ds(0, 128)], buf.at[pl.ds(0, 128)], sem).wait()
    pltpu.make_async_copy(
        p_hbm.at[pl.ds(0, 128)], buf.at[pl.ds(128, 128)], sem).wait()


def _reduce_into(buf, acc_v, p):
    def red(i, accs):
        a0, a1 = accs
        r = i * 8
        sb = plsc.bitcast(buf[r, pl.ds(0, 16)], jnp.bfloat16)
        for k in range(1, 8):
            sb = sb + plsc.bitcast(buf[r + k, pl.ds(0, 16)], jnp.bfloat16)
        lo, hi = plsc.unpack(
            sb, format=plsc.PackFormat.INTERLEAVED,
            preferred_element_type=jnp.float32)
        a0 = a0 + lo
        a1 = a1 + hi
        return (a0, a1)

    z = jnp.zeros((16,), jnp.float32)
    a0, a1 = lax.fori_loop(0, _C // 8, red, (z, z))
    b1 = p // _T
    t = p - b1 * _T
    acc_v[b1, t, pl.ds(0, 16)] = a0
    acc_v[b1, t, pl.ds(16, 16)] = a1


def _sc_body(p_hbm, x_hbm, out_hbm, idx_v, bufs, acc_v, sems):
    wid = lax.axis_index("s") * _NC + lax.axis_index("c")
    for j2 in range(_BPW):
        pltpu.sync_copy(
            x_hbm.at[wid * _BPW + j2], idx_v.at[pl.ds(j2 * _T, _T)])

    lane = lax.iota(jnp.int32, 16)
    patt = (lane // 8) * 2048 + (lane % 8)

    def xform(r, _):
        for j in range(_C // 16):
            v = idx_v[r, pl.ds(16 * j, 16)]
            idx_v[r, pl.ds(16 * j, 16)] = v * 8 + (patt + 4096 * j)
        return 0

    lax.fori_loop(0, _PPW, xform, 0)

    for b in range(_NBUF - 1):
        _issue(p_hbm, idx_v, b, bufs[b], sems[b])

    def ring(i, _):
        p0 = i * _NBUF
        for b in range(_NBUF):
            p = p0 + b
            _drain(p_hbm, bufs[b], sems[b])
            _reduce_into(bufs[b], acc_v, p)

            @pl.when(p + _NBUF - 1 < _PPW)
            def _(p=p, b=b):
                _issue(p_hbm, idx_v, p + _NBUF - 1,
                       bufs[(b + _NBUF - 1) % _NBUF],
                       sems[(b + _NBUF - 1) % _NBUF])
        return 0

    lax.fori_loop(0, _PPW // _NBUF, ring, 0)
    pltpu.sync_copy(acc_v, out_hbm.at[pl.ds(wid * _BPW, _BPW)])


@functools.partial(
    pl.kernel,
    mesh=plsc.VectorSubcoreMesh(core_axis_name="c", subcore_axis_name="s"),
    compiler_params=pltpu.CompilerParams(
        use_tc_tiling_on_sc=False, needs_layout_passes=False),
    out_type=jax.ShapeDtypeStruct((_B, _T, _D), jnp.float32),
    scratch_types=(
        [pltpu.VMEM((_PPW, _C), jnp.int32)]
        + [pltpu.VMEM((_C, _D // 2), jnp.float32) for _ in range(_NBUF)]
        + [pltpu.VMEM((_BPW, _T, _D), jnp.float32)]
        + [pltpu.SemaphoreType.DMA for _ in range(_NBUF)]
    ),
)
def _sc_gather_sum(p_hbm, ids_hbm, out_hbm, idx_v, *rest):
    bufs = rest[:_NBUF]
    acc_v = rest[_NBUF]
    sems = rest[_NBUF + 1:]
    _sc_body(p_hbm, ids_hbm, out_hbm, idx_v, bufs, acc_v, sems)


def kernel(x, table, proj_w):
    ptc = _precompute_p(table, proj_w)
    pflat = ptc.reshape(_C * _C, _D // 2)
    return _sc_gather_sum(pflat, x)


# speedup vs baseline: 63.5224x; 1.0872x over previous
import functools

import jax
import jax.numpy as jnp
from jax import lax
from jax.experimental import pallas as pl
from jax.experimental.pallas import tpu as pltpu
from jax.experimental.pallas import tpu_sc as plsc

_B, _T, _C, _D = 64, 100, 256, 32
_N = _B * _T
_NC, _NS = 2, 16
_NW = _NC * _NS
_PPW = _N // _NW
_BPW = _PPW // _T
_CB = 8
_SUB = 4
_NBUF = 8


def _tc_body(table_ref, pw_ref, out_ref, gw_ref, mask_ref):
    c0 = pl.program_id(0)
    k = _CB * _D

    @pl.when(c0 == 0)
    def _():
        t = table_ref[...]
        g = t * 0.5 * (1.0 + lax.erf(t * (2.0 ** -0.5)))
        gw_ref[...] = jnp.concatenate([g] * _CB, axis=1).astype(jnp.bfloat16)
        ri = lax.broadcasted_iota(jnp.int32, (k, k), 0)
        ci = lax.broadcasted_iota(jnp.int32, (k, k), 1)
        mask_ref[...] = jnp.where(
            (ri // _D) == ((ci % (k // 2)) // 16), 1.0, 0.0)

    gw = gw_ref[...]
    mask = mask_ref[...]
    for s in range(_SUB):
        pw = pw_ref[s * k:(s + 1) * k, :]
        pwwide = jnp.concatenate(
            [pw[:, :16]] * _CB + [pw[:, 16:]] * _CB, axis=1)
        rhs = (pwwide * mask).astype(jnp.bfloat16)
        res = jnp.dot(gw, rhs, preferred_element_type=jnp.float32)
        ri = lax.bitcast_convert_type(res, jnp.int32)
        rnd = lax.shift_right_logical(
            ri + 0x7FFF + (lax.shift_right_logical(ri, 16) & 1), 16)
        words = rnd[:, :k // 2] | (rnd[:, k // 2:] << 16)
        out_ref[s * 256:(s + 1) * 256, :] = lax.bitcast_convert_type(
            words, jnp.float32)


def _precompute_p(table, proj_w):
    steps = _C // (_CB * _SUB)
    return pl.pallas_call(
        _tc_body,
        grid=(steps,),
        in_specs=[
            pl.BlockSpec((_C, _D), lambda c: (0, 0)),
            pl.BlockSpec((_SUB * _CB * _D, _D), lambda c: (c, 0)),
        ],
        out_specs=pl.BlockSpec((_SUB * 256, 128), lambda c: (c, 0)),
        out_shape=jax.ShapeDtypeStruct((_C * _C // 8, 128), jnp.float32),
        scratch_shapes=[
            pltpu.VMEM((_C, _CB * _D), jnp.bfloat16),
            pltpu.VMEM((_CB * _D, _CB * _D), jnp.float32),
        ],
    )(table, proj_w)


def _issue(p_hbm, idx_v, p, buf, sem):
    pltpu.async_copy(
        p_hbm.at[idx_v.at[p, pl.ds(0, 128)]], buf.at[pl.ds(0, 128)], sem)
    pltpu.async_copy(
        p_hbm.at[idx_v.at[p, pl.ds(128, 128)]], buf.at[pl.ds(128, 128)], sem)


def _drain(p_hbm, buf, sem):
    pltpu.make_async_copy(
        p_hbm.at[pl.ds(0, 128)], buf.at[pl.ds(0, 128)], sem).wait()
    pltpu.make_async_copy(
        p_hbm.at[pl.ds(0, 128)], buf.at[pl.ds(128, 128)], sem).wait()


def _reduce_into(buf, acc_v, p):
    def red(i, accs):
        a0, a1 = accs
        r = i * 8
        sb = plsc.bitcast(buf[r, pl.ds(0, 16)], jnp.bfloat16)
        for k in range(1, 8):
            sb = sb + plsc.bitcast(buf[r + k, pl.ds(0, 16)], jnp.bfloat16)
        lo, hi = plsc.unpack(
            sb, format=plsc.PackFormat.INTERLEAVED,
            preferred_element_type=jnp.float32)
        a0 = a0 + lo
        a1 = a1 + hi
        return (a0, a1)

    z = jnp.zeros((16,), jnp.float32)
    a0, a1 = lax.fori_loop(0, _C // 8, red, (z, z))
    b1 = p // _T
    t = p - b1 * _T
    acc_v[b1, t, pl.ds(0, 16)] = a0
    acc_v[b1, t, pl.ds(16, 16)] = a1


def _sc_body(p_hbm, x_hbm, out_hbm, idx_v, bufs, acc_v, sems):
    wid = lax.axis_index("s") * _NC + lax.axis_index("c")
    for j2 in range(_BPW):
        pltpu.sync_copy(
            x_hbm.at[wid * _BPW + j2], idx_v.at[pl.ds(j2 * _T, _T)])

    lane = lax.iota(jnp.int32, 16)
    patt = (lane // 8) * 2048 + (lane % 8)

    def xform(r, _):
        for j in range(_C // 16):
            v = idx_v[r, pl.ds(16 * j, 16)]
            idx_v[r, pl.ds(16 * j, 16)] = v * 8 + (patt + 4096 * j)
        return 0

    lax.fori_loop(0, _PPW, xform, 0)

    for b in range(_NBUF - 1):
        _issue(p_hbm, idx_v, b, bufs[b], sems[b])

    def ring(i, _):
        p0 = i * _NBUF
        for b in range(_NBUF):
            p = p0 + b
            _drain(p_hbm, bufs[b], sems[b])
            _reduce_into(bufs[b], acc_v, p)

            @pl.when(p + _NBUF - 1 < _PPW)
            def _(p=p, b=b):
                _issue(p_hbm, idx_v, p + _NBUF - 1,
                       bufs[(b + _NBUF - 1) % _NBUF],
                       sems[(b + _NBUF - 1) % _NBUF])
        return 0

    lax.fori_loop(0, _PPW // _NBUF, ring, 0)
    pltpu.sync_copy(acc_v, out_hbm.at[pl.ds(wid * _BPW, _BPW)])


@functools.partial(
    pl.kernel,
    mesh=plsc.VectorSubcoreMesh(core_axis_name="c", subcore_axis_name="s"),
    compiler_params=pltpu.CompilerParams(
        use_tc_tiling_on_sc=False, needs_layout_passes=False),
    out_type=jax.ShapeDtypeStruct((_B, _T, _D), jnp.float32),
    scratch_types=(
        [pltpu.VMEM((_PPW, _C), jnp.int32)]
        + [pltpu.VMEM((_C, _D // 2), jnp.float32) for _ in range(_NBUF)]
        + [pltpu.VMEM((_BPW, _T, _D), jnp.float32)]
        + [pltpu.SemaphoreType.DMA for _ in range(_NBUF)]
    ),
)
def _sc_gather_sum(p_hbm, ids_hbm, out_hbm, idx_v, *rest):
    bufs = rest[:_NBUF]
    acc_v = rest[_NBUF]
    sems = rest[_NBUF + 1:]
    _sc_body(p_hbm, ids_hbm, out_hbm, idx_v, bufs, acc_v, sems)


def kernel(x, table, proj_w):
    ptc = _precompute_p(table, proj_w)
    pflat = ptc.reshape(_C * _C, _D // 2)
    return _sc_gather_sum(pflat, x)


# speedup vs baseline: 63.7021x; 1.0028x over previous
import functools

import jax
import jax.numpy as jnp
from jax import lax
from jax.experimental import pallas as pl
from jax.experimental.pallas import tpu as pltpu
from jax.experimental.pallas import tpu_sc as plsc

_B, _T, _C, _D = 64, 100, 256, 32
_N = _B * _T
_NC, _NS = 2, 16
_NW = _NC * _NS
_PPW = _N // _NW
_BPW = _PPW // _T
_CB = 8
_SUB = 4
_NBUF = 8


def _tc_body(table_ref, pw_ref, out_ref, gw_ref, mask_ref):
    c0 = pl.program_id(0)
    k = _CB * _D

    @pl.when(c0 == 0)
    def _():
        t = table_ref[...]
        g = t * 0.5 * (1.0 + lax.erf(t * (2.0 ** -0.5)))
        gw_ref[...] = jnp.concatenate([g] * _CB, axis=1).astype(jnp.bfloat16)
        ri = lax.broadcasted_iota(jnp.int32, (k, k), 0)
        ci = lax.broadcasted_iota(jnp.int32, (k, k), 1)
        mask_ref[...] = jnp.where(
            (ri // _D) == ((ci % (k // 2)) // 16), 1.0, 0.0)

    gw = gw_ref[...]
    mask = mask_ref[...]
    for s in range(_SUB):
        pw = pw_ref[s * k:(s + 1) * k, :]
        pwwide = jnp.concatenate(
            [pw[:, :16]] * _CB + [pw[:, 16:]] * _CB, axis=1)
        rhs = (pwwide * mask).astype(jnp.bfloat16)
        res = jnp.dot(gw, rhs, preferred_element_type=jnp.float32)
        ri = lax.bitcast_convert_type(res, jnp.int32)
        rnd = lax.shift_right_logical(
            ri + 0x7FFF + (lax.shift_right_logical(ri, 16) & 1), 16)
        words = rnd[:, :k // 2] | (rnd[:, k // 2:] << 16)
        out_ref[s * 256:(s + 1) * 256, :] = lax.bitcast_convert_type(
            words, jnp.float32)


def _precompute_p(table, proj_w):
    steps = _C // (_CB * _SUB)
    return pl.pallas_call(
        _tc_body,
        grid=(steps,),
        in_specs=[
            pl.BlockSpec((_C, _D), lambda c: (0, 0)),
            pl.BlockSpec((_SUB * _CB * _D, _D), lambda c: (c, 0)),
        ],
        out_specs=pl.BlockSpec((_SUB * 256, 128), lambda c: (c, 0)),
        out_shape=jax.ShapeDtypeStruct((_C * _C // 8, 128), jnp.float32),
        scratch_shapes=[
            pltpu.VMEM((_C, _CB * _D), jnp.bfloat16),
            pltpu.VMEM((_CB * _D, _CB * _D), jnp.float32),
        ],
    )(table, proj_w)


def _issue(p_hbm, idx_v, p, buf, sem):
    pltpu.async_copy(
        p_hbm.at[idx_v.at[pl.ds(p * _C, 128)]], buf.at[pl.ds(0, 128)], sem)
    pltpu.async_copy(
        p_hbm.at[idx_v.at[pl.ds(p * _C + 128, 128)]],
        buf.at[pl.ds(128, 128)], sem)


def _drain(p_hbm, buf, sem):
    pltpu.make_async_copy(
        p_hbm.at[pl.ds(0, 128)], buf.at[pl.ds(0, 128)], sem).wait()
    pltpu.make_async_copy(
        p_hbm.at[pl.ds(0, 128)], buf.at[pl.ds(128, 128)], sem).wait()


def _reduce_into(buf, acc_v, p):
    def red(i, accs):
        a0, a1 = accs
        r = i * 8
        sb = plsc.bitcast(buf[r, pl.ds(0, 16)], jnp.bfloat16)
        for k in range(1, 8):
            sb = sb + plsc.bitcast(buf[r + k, pl.ds(0, 16)], jnp.bfloat16)
        lo, hi = plsc.unpack(
            sb, format=plsc.PackFormat.INTERLEAVED,
            preferred_element_type=jnp.float32)
        a0 = a0 + lo
        a1 = a1 + hi
        return (a0, a1)

    z = jnp.zeros((16,), jnp.float32)
    a0, a1 = lax.fori_loop(0, _C // 8, red, (z, z))
    b1 = p // _T
    t = p - b1 * _T
    acc_v[b1, t, pl.ds(0, 16)] = a0
    acc_v[b1, t, pl.ds(16, 16)] = a1


def _sc_body(p_hbm, x_hbm, out_hbm, idx_v, bufs, acc_v, sems):
    wid = lax.axis_index("s") * _NC + lax.axis_index("c")
    pltpu.sync_copy(
        x_hbm.at[pl.ds(wid * _PPW * _C, _PPW * _C)], idx_v)

    lane = lax.iota(jnp.int32, 16)
    patt = (lane // 8) * 2048 + (lane % 8)

    def xform(r, _):
        for j in range(_C // 16):
            v = idx_v[pl.ds(r * _C + 16 * j, 16)]
            idx_v[pl.ds(r * _C + 16 * j, 16)] = v * 8 + (patt + 4096 * j)
        return 0

    lax.fori_loop(0, _PPW, xform, 0)

    for b in range(_NBUF - 1):
        _issue(p_hbm, idx_v, b, bufs[b], sems[b])

    def ring(i, _):
        p0 = i * _NBUF
        for b in range(_NBUF):
            p = p0 + b
            _drain(p_hbm, bufs[b], sems[b])
            _reduce_into(bufs[b], acc_v, p)

            @pl.when(p + _NBUF - 1 < _PPW)
            def _(p=p, b=b):
                _issue(p_hbm, idx_v, p + _NBUF - 1,
                       bufs[(b + _NBUF - 1) % _NBUF],
                       sems[(b + _NBUF - 1) % _NBUF])
        return 0

    lax.fori_loop(0, _PPW // _NBUF, ring, 0)
    pltpu.sync_copy(acc_v, out_hbm.at[pl.ds(wid * _BPW, _BPW)])


@functools.partial(
    pl.kernel,
    mesh=plsc.VectorSubcoreMesh(core_axis_name="c", subcore_axis_name="s"),
    compiler_params=pltpu.CompilerParams(
        use_tc_tiling_on_sc=False, needs_layout_passes=False),
    out_type=jax.ShapeDtypeStruct((_B, _T, _D), jnp.float32),
    scratch_types=(
        [pltpu.VMEM((_PPW * _C,), jnp.int32)]
        + [pltpu.VMEM((_C, _D // 2), jnp.float32) for _ in range(_NBUF)]
        + [pltpu.VMEM((_BPW, _T, _D), jnp.float32)]
        + [pltpu.SemaphoreType.DMA for _ in range(_NBUF)]
    ),
)
def _sc_gather_sum(p_hbm, ids_hbm, out_hbm, idx_v, *rest):
    bufs = rest[:_NBUF]
    acc_v = rest[_NBUF]
    sems = rest[_NBUF + 1:]
    _sc_body(p_hbm, ids_hbm, out_hbm, idx_v, bufs, acc_v, sems)


def kernel(x, table, proj_w):
    ptc = _precompute_p(table, proj_w)
    pflat = ptc.reshape(_C * _C, _D // 2)
    return _sc_gather_sum(pflat, x.reshape(-1))
